# baseline (speedup 1.0000x reference)
import jax
import jax.numpy as jnp
from jax import lax
from jax.experimental import pallas as pl
from jax.experimental.pallas import tpu as pltpu
from jax.experimental.pallas import tpu_sc as plsc

N = 8192
M = 8192
C = 16384
THRESH = 0.1

L = 16
NW = 16
CHUNK = C // NW
BAND = N // NW
GCH = 128
NG = CHUNK // GCH
VPG = GCH // L
ZU = 8
DUMMY = N * M
LN2 = 0.6931471805599453


def _neg_log(t):
    bits = lax.bitcast_convert_type(t, jnp.int32)
    e = (bits >> 23) - 127
    m = lax.bitcast_convert_type(
        (bits & 0x007FFFFF) | 0x3F800000, jnp.float32)
    r = (m - 1.0) / (m + 1.0)
    r2 = r * r
    p = 1.0 + r2 * (1.0 / 3.0 + r2 * (1.0 / 5.0 + r2 * (1.0 / 7.0 + r2 / 9.0)))
    return -(e.astype(jnp.float32) * LN2 + 2.0 * r * p)


def _sc_body(scores_hbm, rows_hbm, cols_hbm, vals_hbm, out_hbm, tag_hbm,
             r_v, c_v, v_v, lrm, lcm, band_r, band_red,
             rm_all, cm_all, rm_sh, cm_sh, part_sh,
             flat2d, tag2d, pnd2d, tagg_v, cval_v, s_v,
             part_v, parts_v, out_v, sem_g, sem_d):
    w = lax.axis_index("s")
    base = w * CHUNK

    stage = [pltpu.async_copy(rows_hbm.at[pl.ds(base, CHUNK)], r_v, sem_d),
             pltpu.async_copy(cols_hbm.at[pl.ds(base, CHUNK)], c_v, sem_d),
             pltpu.async_copy(vals_hbm.at[pl.ds(base, CHUNK)], v_v, sem_d)]

    iota = lax.iota(jnp.int32, L)
    zerosf = jnp.zeros((L,), jnp.float32)

    def zbody(i, _):
        for u in range(ZU):
            lrm[pl.ds((i * ZU + u) * L, L)] = zerosf
            lcm[pl.ds((i * ZU + u) * L, L)] = zerosf
        return 0
    lax.fori_loop(0, N // L // ZU, zbody, 0)
    for cp in stage:
        cp.wait()

    for g in range(NG):
        def fbody(j, _):
            o = g * GCH + j * L
            rv = r_v[pl.ds(o, L)]
            cv = c_v[pl.ds(o, L)]
            flat2d[g, pl.ds(j * L, L)] = (
                ((rv >> 3) << 16) | ((cv >> 7) << 10) | ((rv & 7) << 7)
                | (cv & 127))
            tag2d[g, pl.ds(j * L, L)] = jnp.full((L,), base + o, jnp.int32) + iota
            return 0
        lax.fori_loop(0, VPG, fbody, 0)

    gathers = [
        pltpu.async_copy(scores_hbm.at[flat2d.at[g]],
                         s_v.at[pl.ds(g * GCH, GCH)], sem_g)
        for g in range(NG)
    ]

    s1 = [pltpu.async_copy(tag2d.at[g], tag_hbm.at[flat2d.at[g]], sem_d)
          for g in range(NG)]
    for cp in s1:
        cp.wait()
    plsc.subcore_barrier()

    def consensus_round():
        g1 = [pltpu.async_copy(tag_hbm.at[flat2d.at[g]],
                               tagg_v.at[pl.ds(g * GCH, GCH)], sem_d)
              for g in range(NG)]
        for cp in g1:
            cp.wait()
        pend_acc = jnp.zeros((L,), jnp.int32)
        for g in range(NG):
            def pbody(j, acc):
                o = g * GCH + j * L
                got = tagg_v[pl.ds(o, L)]
                mine = tag2d[g, pl.ds(j * L, L)]
                flat = flat2d[g, pl.ds(j * L, L)]
                pend = got < mine
                pnd2d[g, pl.ds(j * L, L)] = jnp.where(pend, flat, DUMMY)
                return acc + plsc.all_reduce_population_count(pend)
            pend_acc = lax.fori_loop(0, VPG, pbody, pend_acc)
        npend = pend_acc[0]

        @pl.when(npend > 0)
        def _():
            s2 = [pltpu.async_copy(tag2d.at[g], tag_hbm.at[pnd2d.at[g]], sem_d)
                  for g in range(NG)]
            for cp in s2:
                cp.wait()
        plsc.subcore_barrier()

    consensus_round()
    consensus_round()

    gf = [pltpu.async_copy(tag_hbm.at[flat2d.at[g]],
                           tagg_v.at[pl.ds(g * GCH, GCH)], sem_d)
          for g in range(NG)]
    for cp in gf:
        cp.wait()
    vf = [pltpu.async_copy(vals_hbm.at[tagg_v.at[pl.ds(g * GCH, GCH)]],
                           cval_v.at[pl.ds(g * GCH, GCH)], sem_d)
          for g in range(NG)]
    for cp in vf:
        cp.wait()

    def rmw_max(table, iv, vv):
        k, v = plsc.sort_key_val(iv, vv)
        for d in (1, 2, 4, 8):
            srci = jnp.maximum(iota - d, 0)
            ks = k.at[srci].get(mode="promise_in_bounds")
            vs = v.at[srci].get(mode="promise_in_bounds")
            same = (ks == k) & (iota >= d)
            v = jnp.where(same, jnp.maximum(v, vs), v)
        nxt = jnp.minimum(iota + 1, L - 1)
        kn = k.at[nxt].get(mode="promise_in_bounds")
        is_last = (k != kn) | (iota == L - 1)
        cur = plsc.load_gather(table, [k], mask=is_last)
        plsc.store_scatter(table, [k], jnp.maximum(v, cur), mask=is_last)

    def sbody(j, _):
        rv = r_v[pl.ds(j * L, L)]
        cv = c_v[pl.ds(j * L, L)]
        vv = cval_v[pl.ds(j * L, L)]
        rmw_max(lrm, rv, vv)
        rmw_max(lcm, cv, vv)
        return 0
    lax.fori_loop(0, CHUNK // L, sbody, 0)

    pltpu.sync_copy(lrm, rm_all.at[w])
    pltpu.sync_copy(lcm, cm_all.at[w])
    plsc.subcore_barrier()

    def reduce_band(all_sh, final_sh):
        band_cps = [
            pltpu.async_copy(
                all_sh.at[u, pl.ds(w * BAND, BAND)], band_r.at[u], sem_d)
            for u in range(NW)
        ]
        for cp in band_cps:
            cp.wait()

        def rbody(j, _):
            acc = band_r[0, pl.ds(j * L, L)]
            for u in range(1, NW):
                acc = jnp.maximum(acc, band_r[u, pl.ds(j * L, L)])
            band_red[pl.ds(j * L, L)] = acc
            return 0
        lax.fori_loop(0, BAND // L, rbody, 0)
        pltpu.sync_copy(band_red, final_sh.at[pl.ds(w * BAND, BAND)])

    reduce_band(rm_all, rm_sh)
    reduce_band(cm_all, cm_sh)
    plsc.subcore_barrier()

    pltpu.sync_copy(rm_sh, lrm)
    pltpu.sync_copy(cm_sh, lcm)

    for cp in gathers:
        cp.wait()

    def selbody(j, accs):
        num_acc, den_acc = accs
        rv = r_v[pl.ds(j * L, L)]
        cv = c_v[pl.ds(j * L, L)]
        vv = cval_v[pl.ds(j * L, L)]
        sv = s_v[pl.ds(j * L, L)]
        mine = jnp.full((L,), base, jnp.int32) + j * L + iota
        won = tagg_v[pl.ds(j * L, L)] == mine
        rm = plsc.load_gather(lrm, [rv])
        cm = plsc.load_gather(lcm, [cv])
        sel = won & (vv > THRESH) & (vv == rm) & (vv == cm)
        mv = jnp.where(sel, vv, 0.0)
        num_acc = num_acc + mv * _neg_log(sv + 1e-8)
        den_acc = den_acc + mv
        return num_acc, den_acc
    num_acc, den_acc = lax.fori_loop(
        0, CHUNK // L, selbody, (zerosf, zerosf))

    num_s = jnp.sum(num_acc)
    den_s = jnp.sum(den_acc)
    part_v[...] = jnp.where(iota == 0, num_s, jnp.where(iota == 1, den_s, 0.0))
    pltpu.sync_copy(part_v, part_sh.at[w])
    plsc.subcore_barrier()

    @pl.when(w == 0)
    def _():
        pltpu.sync_copy(part_sh, parts_v)
        tot = parts_v[0, :]
        for u in range(1, NW):
            tot = tot + parts_v[u, :]
        nxt = jnp.minimum(iota + 1, L - 1)
        den_vec = tot.at[nxt].get(mode="promise_in_bounds")
        out_v[...] = tot / den_vec
        pltpu.sync_copy(out_v, out_hbm)


def _sc_stage(scores_phys, rows, cols, vals):
    mesh = plsc.VectorSubcoreMesh(
        core_axis_name="c", subcore_axis_name="s", num_cores=1)
    f32 = jnp.float32
    run = pl.kernel(
        _sc_body,
        out_type=(jax.ShapeDtypeStruct((L,), f32),
                  jax.ShapeDtypeStruct((N * M + 64,), jnp.int32)),
        mesh=mesh,
        compiler_params=pltpu.CompilerParams(
            needs_layout_passes=False, use_tc_tiling_on_sc=True),
        scratch_types=[
            pltpu.VMEM((CHUNK,), jnp.int32),
            pltpu.VMEM((CHUNK,), jnp.int32),
            pltpu.VMEM((CHUNK,), f32),
            pltpu.VMEM((N,), f32),
            pltpu.VMEM((M,), f32),
            pltpu.VMEM((NW, BAND), f32),
            pltpu.VMEM((BAND,), f32),
            pltpu.MemorySpace.VMEM_SHARED((NW, N), f32),
            pltpu.MemorySpace.VMEM_SHARED((NW, M), f32),
            pltpu.MemorySpace.VMEM_SHARED((N,), f32),
            pltpu.MemorySpace.VMEM_SHARED((M,), f32),
            pltpu.MemorySpace.VMEM_SHARED((NW, L), f32),
            pltpu.VMEM((NG, GCH), jnp.int32),
            pltpu.VMEM((NG, GCH), jnp.int32),
            pltpu.VMEM((NG, GCH), jnp.int32),
            pltpu.VMEM((CHUNK,), jnp.int32),
            pltpu.VMEM((CHUNK,), f32),
            pltpu.VMEM((CHUNK,), f32),
            pltpu.VMEM((L,), f32),
            pltpu.VMEM((NW, L), f32),
            pltpu.VMEM((L,), f32),
            pltpu.SemaphoreType.DMA,
            pltpu.SemaphoreType.DMA,
        ],
    )
    return run(scores_phys, rows, cols, vals)


def kernel(coarse_matching_scores, gt_patch_corr_indices, gt_patch_corr_overlaps):
    rows = gt_patch_corr_indices[:, 0]
    cols = gt_patch_corr_indices[:, 1]
    scores_phys = coarse_matching_scores.reshape(
        N // 8, 8, M // 128, 128).transpose(0, 2, 1, 3).reshape(-1)
    out, _ = _sc_stage(scores_phys, rows, cols, gt_patch_corr_overlaps)
    return out[0]


# speedup vs baseline: 6.7172x; 6.7172x over previous
import jax
import jax.numpy as jnp
from jax import lax
from jax.experimental import pallas as pl
from jax.experimental.pallas import tpu as pltpu
from jax.experimental.pallas import tpu_sc as plsc

N = 8192
M = 8192
C = 16384
THRESH = 0.1

L = 16
NW = 16
CHUNK = C // NW
BAND = N // NW
GCH = 128
NG = CHUNK // GCH
ZU = 8
LN2 = 0.6931471805599453


def _neg_log(t):
    bits = lax.bitcast_convert_type(t, jnp.int32)
    e = (bits >> 23) - 127
    m = lax.bitcast_convert_type(
        (bits & 0x007FFFFF) | 0x3F800000, jnp.float32)
    r = (m - 1.0) / (m + 1.0)
    r2 = r * r
    p = 1.0 + r2 * (1.0 / 3.0 + r2 * (1.0 / 5.0 + r2 * (1.0 / 7.0 + r2 / 9.0)))
    return -(e.astype(jnp.float32) * LN2 + 2.0 * r * p)


def _sc_body(scores_hbm, rows_hbm, cols_hbm, vals_hbm, out_hbm,
             r_v, c_v, v_v, lrm, lcm, band_r, band_red,
             rm_all, cm_all, rm_sh, cm_sh, part_sh,
             flat_v, s_v, part_v, parts_v, out_v, sem_g, sem_d):
    w = lax.axis_index("s")
    base = w * CHUNK

    stage = [pltpu.async_copy(rows_hbm.at[pl.ds(base, CHUNK)], r_v, sem_d),
             pltpu.async_copy(cols_hbm.at[pl.ds(base, CHUNK)], c_v, sem_d),
             pltpu.async_copy(vals_hbm.at[pl.ds(base, CHUNK)], v_v, sem_d)]

    iota = lax.iota(jnp.int32, L)

    zerosf = jnp.zeros((L,), jnp.float32)

    def zbody(i, _):
        for u in range(ZU):
            lrm[pl.ds((i * ZU + u) * L, L)] = zerosf
            lcm[pl.ds((i * ZU + u) * L, L)] = zerosf
        return 0
    lax.fori_loop(0, N // L // ZU, zbody, 0)
    for cp in stage:
        cp.wait()

    def rmw_max(table, iv, vv):
        k, v = plsc.sort_key_val(iv, vv)
        for d in (1, 2, 4, 8):
            srci = jnp.maximum(iota - d, 0)
            ks = k.at[srci].get(mode="promise_in_bounds")
            vs = v.at[srci].get(mode="promise_in_bounds")
            same = (ks == k) & (iota >= d)
            v = jnp.where(same, jnp.maximum(v, vs), v)
        nxt = jnp.minimum(iota + 1, L - 1)
        kn = k.at[nxt].get(mode="promise_in_bounds")
        is_last = (k != kn) | (iota == L - 1)
        cur = plsc.load_gather(table, [k], mask=is_last)
        plsc.store_scatter(table, [k], jnp.maximum(v, cur), mask=is_last)

    def sbody(j, _):
        rv = r_v[pl.ds(j * L, L)]
        cv = c_v[pl.ds(j * L, L)]
        vv = v_v[pl.ds(j * L, L)]
        flat_v[pl.ds(j * L, L)] = (
            ((rv >> 3) << 16) | ((cv >> 7) << 10) | ((rv & 7) << 7) | (cv & 127))
        rmw_max(lrm, rv, vv)
        rmw_max(lcm, cv, vv)
        return 0
    lax.fori_loop(0, CHUNK // L, sbody, 0)

    gathers = [
        pltpu.async_copy(scores_hbm.at[flat_v.at[pl.ds(k * GCH, GCH)]],
                         s_v.at[pl.ds(k * GCH, GCH)], sem_g)
        for k in range(NG)
    ]

    pltpu.sync_copy(lrm, rm_all.at[w])
    pltpu.sync_copy(lcm, cm_all.at[w])
    plsc.subcore_barrier()

    def reduce_band(all_sh, final_sh):
        band_cps = [
            pltpu.async_copy(
                all_sh.at[u, pl.ds(w * BAND, BAND)], band_r.at[u], sem_d)
            for u in range(NW)
        ]
        for cp in band_cps:
            cp.wait()

        def rbody(j, _):
            acc = band_r[0, pl.ds(j * L, L)]
            for u in range(1, NW):
                acc = jnp.maximum(acc, band_r[u, pl.ds(j * L, L)])
            band_red[pl.ds(j * L, L)] = acc
            return 0
        lax.fori_loop(0, BAND // L, rbody, 0)
        pltpu.sync_copy(band_red, final_sh.at[pl.ds(w * BAND, BAND)])

    reduce_band(rm_all, rm_sh)
    reduce_band(cm_all, cm_sh)
    plsc.subcore_barrier()

    pltpu.sync_copy(rm_sh, lrm)
    pltpu.sync_copy(cm_sh, lcm)
    rm_v = lrm
    cm_v = lcm

    for cp in gathers:
        cp.wait()

    def selbody(j, accs):
        num_acc, den_acc = accs
        rv = r_v[pl.ds(j * L, L)]
        cv = c_v[pl.ds(j * L, L)]
        vv = v_v[pl.ds(j * L, L)]
        sv = s_v[pl.ds(j * L, L)]
        rm = plsc.load_gather(rm_v, [rv])
        cm = plsc.load_gather(cm_v, [cv])
        sel = (vv > THRESH) & (vv == rm) & (vv == cm)
        mv = jnp.where(sel, vv, 0.0)
        num_acc = num_acc + mv * _neg_log(sv + 1e-8)
        den_acc = den_acc + mv
        return num_acc, den_acc
    num_acc, den_acc = lax.fori_loop(
        0, CHUNK // L, selbody, (zerosf, zerosf))

    num_s = jnp.sum(num_acc)
    den_s = jnp.sum(den_acc)
    part_v[...] = jnp.where(iota == 0, num_s, jnp.where(iota == 1, den_s, 0.0))
    pltpu.sync_copy(part_v, part_sh.at[w])
    plsc.subcore_barrier()

    @pl.when(w == 0)
    def _():
        pltpu.sync_copy(part_sh, parts_v)
        tot = parts_v[0, :]
        for u in range(1, NW):
            tot = tot + parts_v[u, :]
        nxt = jnp.minimum(iota + 1, L - 1)
        den_vec = tot.at[nxt].get(mode="promise_in_bounds")
        out_v[...] = tot / den_vec
        pltpu.sync_copy(out_v, out_hbm)


def _sc_stage(scores_phys, rows, cols, vals):
    mesh = plsc.VectorSubcoreMesh(
        core_axis_name="c", subcore_axis_name="s", num_cores=1)
    f32 = jnp.float32
    run = pl.kernel(
        _sc_body,
        out_type=jax.ShapeDtypeStruct((L,), f32),
        mesh=mesh,
        compiler_params=pltpu.CompilerParams(
            needs_layout_passes=False, use_tc_tiling_on_sc=True),
        scratch_types=[
            pltpu.VMEM((CHUNK,), jnp.int32),
            pltpu.VMEM((CHUNK,), jnp.int32),
            pltpu.VMEM((CHUNK,), f32),
            pltpu.VMEM((N,), f32),
            pltpu.VMEM((M,), f32),
            pltpu.VMEM((NW, BAND), f32),
            pltpu.VMEM((BAND,), f32),
            pltpu.MemorySpace.VMEM_SHARED((NW, N), f32),
            pltpu.MemorySpace.VMEM_SHARED((NW, M), f32),
            pltpu.MemorySpace.VMEM_SHARED((N,), f32),
            pltpu.MemorySpace.VMEM_SHARED((M,), f32),
            pltpu.MemorySpace.VMEM_SHARED((NW, L), f32),
            pltpu.VMEM((CHUNK,), jnp.int32),
            pltpu.VMEM((CHUNK,), f32),
            pltpu.VMEM((L,), f32),
            pltpu.VMEM((NW, L), f32),
            pltpu.VMEM((L,), f32),
            pltpu.SemaphoreType.DMA,
            pltpu.SemaphoreType.DMA,
        ],
    )
    return run(scores_phys, rows, cols, vals)


def kernel(coarse_matching_scores, gt_patch_corr_indices, gt_patch_corr_overlaps):
    scores_phys = coarse_matching_scores.reshape(
        N // 8, 8, M // 128, 128).transpose(0, 2, 1, 3).reshape(-1)
    rows = gt_patch_corr_indices[:, 0]
    cols = gt_patch_corr_indices[:, 1]
    out = _sc_stage(scores_phys, rows, cols, gt_patch_corr_overlaps)
    return out[0]
